# Initial kernel scaffold; baseline (speedup 1.0000x reference)
import jax
import jax.numpy as jnp
from jax.experimental import pallas as pl


def kernel(seq, weight):
    raise NotImplementedError("write your pallas kernel here")


# speedup vs baseline: 1.4681x; 1.4681x over previous
import functools

import jax
import jax.numpy as jnp
from jax import lax
from jax.experimental import pallas as pl
from jax.experimental.pallas import tpu as pltpu
from jax.experimental.pallas import tpu_sc as plsc

VOCAB = 1000000
EMBED = 32
ROWS = 4096
COLS = 200
TOTAL = ROWS * COLS

NC = 2
NS = 16
NW = NC * NS
PER_W = TOTAL // NW
CHUNK = 1280
NCHUNK = PER_W // CHUNK


def _embed_body(idx_hbm, table_hbm, out_hbm, idx_v, rows_v, sem_g):
    c = lax.axis_index("c")
    s = lax.axis_index("s")
    wid = s * NC + c
    base = wid * PER_W
    for i in range(NCHUNK):
        off = base + i * CHUNK
        pltpu.sync_copy(idx_hbm.at[pl.ds(off, CHUNK)], idx_v)
        pltpu.async_copy(table_hbm.at[idx_v], rows_v, sem_g).wait()
        pltpu.sync_copy(rows_v, out_hbm.at[pl.ds(off, CHUNK)])


@jax.jit
def _embed(idx_flat, weight):
    mesh = plsc.VectorSubcoreMesh(core_axis_name="c", subcore_axis_name="s")
    return pl.kernel(
        _embed_body,
        mesh=mesh,
        out_type=jax.ShapeDtypeStruct((TOTAL, EMBED), jnp.float32),
        scratch_types=[
            pltpu.VMEM((CHUNK,), jnp.int32),
            pltpu.VMEM((CHUNK, EMBED), jnp.float32),
            pltpu.SemaphoreType.DMA,
        ],
        compiler_params=pltpu.CompilerParams(use_tc_tiling_on_sc=False),
    )(idx_flat, weight)


def kernel(seq, weight):
    idx_flat = seq.reshape(TOTAL).astype(jnp.int32)
    out = _embed(idx_flat, weight)
    return out.reshape(ROWS, COLS, EMBED)


# speedup vs baseline: 1.5029x; 1.0237x over previous
import jax
import jax.numpy as jnp
from jax import lax
from jax.experimental import pallas as pl
from jax.experimental.pallas import tpu as pltpu
from jax.experimental.pallas import tpu_sc as plsc

VOCAB = 1000000
EMBED = 32
ROWS = 4096
COLS = 200
TOTAL = ROWS * COLS

NC = 2
NS = 16
NW = NC * NS
PER_W = TOTAL // NW
CHUNK = 800
NCHUNK = PER_W // CHUNK
NB = 4
NOUT = NCHUNK // NB


def _embed_body(idx_hbm, table_hbm, out_hbm, idx_v, rows_v, gsem, ssem):
    c = lax.axis_index("c")
    s = lax.axis_index("s")
    wid = s * NC + c
    base = wid * PER_W

    pltpu.sync_copy(idx_hbm.at[pl.ds(base, PER_W)], idx_v)

    def gather_desc(j, b):
        return pltpu.make_async_copy(
            table_hbm.at[idx_v.at[pl.ds(j * CHUNK, CHUNK)]],
            rows_v.at[b],
            gsem.at[b],
        )

    def store_desc(j, b):
        return pltpu.make_async_copy(
            rows_v.at[b],
            out_hbm.at[pl.ds(base + j * CHUNK, CHUNK)],
            ssem.at[b],
        )

    for b in range(NB):
        gather_desc(b, b).start()

    def steady(o, carry):
        for b in range(NB):
            j = o * NB + b
            gather_desc(j, b).wait()
            store_desc(j, b).start()
            store_desc(j, b).wait()
            gather_desc(j + NB, b).start()
        return carry

    lax.fori_loop(0, NOUT - 1, steady, 0)

    for b in range(NB):
        j = (NOUT - 1) * NB + b
        gather_desc(j, b).wait()
        store_desc(j, b).start()
    for b in range(NB):
        j = (NOUT - 1) * NB + b
        store_desc(j, b).wait()


@jax.jit
def _embed(idx_flat, weight):
    mesh = plsc.VectorSubcoreMesh(core_axis_name="c", subcore_axis_name="s")
    return pl.kernel(
        _embed_body,
        mesh=mesh,
        out_type=jax.ShapeDtypeStruct((TOTAL, EMBED), jnp.float32),
        scratch_types=[
            pltpu.VMEM((PER_W,), jnp.int32),
            pltpu.VMEM((NB, CHUNK, EMBED), jnp.float32),
            pltpu.SemaphoreType.DMA((NB,)),
            pltpu.SemaphoreType.DMA((NB,)),
        ],
        compiler_params=pltpu.CompilerParams(use_tc_tiling_on_sc=False),
    )(idx_flat, weight)


def kernel(seq, weight):
    idx_flat = seq.reshape(TOTAL).astype(jnp.int32)
    out = _embed(idx_flat, weight)
    return out.reshape(ROWS, COLS, EMBED)
